# baseline (speedup 1.0000x reference)
import jax, jax.numpy as jnp
from jax.experimental import pallas as pl

EPS = 1e-5
K_STATIC = 30

def _mm(W, x):
    return jnp.einsum('oi,bin->bon', W.astype(jnp.bfloat16), x.astype(jnp.bfloat16),
                      preferred_element_type=jnp.float32)

def kernel(f_sem, f_ins, W_sem, b_sem, g_sem, be_sem, W_ad, b_ad, g_ad, be_ad,
           W_ins, b_ins, g_ins, be_ins, W_sp, b_sp, W_ie, b_ie, k):
    def _mlp1d(x, W, b, gamma, beta):
        y = _mm(W, x) + b[None, :, None]
        y = y / jnp.sqrt(1.0 + EPS) * gamma[None, :, None] + beta[None, :, None]
        return jax.nn.relu(y)
    fs = _mlp1d(f_sem, W_sem, b_sem, g_sem, be_sem)
    adapted = _mlp1d(fs, W_ad, b_ad, g_ad, be_ad)
    fi = _mlp1d(f_ins, W_ins, b_ins, g_ins, be_ins)
    f_sins = fi + adapted
    e_ins = _mm(W_ie, f_sins) + b_ie[None, :, None]
    eb = e_ins.astype(jnp.bfloat16)
    inner = jnp.einsum('bcn,bcm->bnm', eb, eb, preferred_element_type=jnp.float32)
    sq = jnp.sum(e_ins * e_ins, axis=1)
    dist = sq[:, :, None] + sq[:, None, :] - 2.0 * inner
    _, idx = jax.lax.top_k(-dist, K_STATIC)
    k_f_sem = jax.vmap(lambda ff, ii: jnp.take(ff, ii, axis=1))(fs, idx)
    f_isem = jnp.max(k_f_sem, axis=3)
    p_sem = _mm(W_sp, f_isem) + b_sp[None, :, None]
    return (p_sem, e_ins)


# speedup vs baseline: 3.4043x; 3.4043x over previous
import functools
import jax
import jax.numpy as jnp
from jax.experimental import pallas as pl
from jax.experimental.pallas import tpu as pltpu

EPS = 1e-5
K_STATIC = 30
KPAD = 32
LEVELS = 6
TR = 32
CN = 1024
INF = float('inf')
IMAX = 2**31 - 1


def _bf(x):
    return x.astype(jnp.bfloat16)


def _knn_kernel(e8_ref, e8t_ref, idx_out):
    b = pl.program_id(0)
    e8 = e8_ref[0]
    N = e8.shape[1]
    eb = e8.astype(jnp.bfloat16)
    sq = e8[0:1] * e8[0:1]
    for c in range(1, 5):
        sq = sq + e8[c:c+1] * e8[c:c+1]
    et = e8t_ref[0]
    ebt = et.astype(jnp.bfloat16)
    sqr = et[:, 0:1] * et[:, 0:1]
    for c in range(1, 5):
        sqr = sqr + et[:, c:c+1] * et[:, c:c+1]

    mv = [jnp.full((TR, 128), INF, jnp.float32) for _ in range(LEVELS)]
    mi = [jnp.full((TR, 128), IMAX, jnp.int32) for _ in range(LEVELS)]
    lane = jax.lax.broadcasted_iota(jnp.int32, (TR, 128), 1)

    for c in range(N // CN):
        ec = eb[:, c * CN:(c + 1) * CN]
        inner = jax.lax.dot_general(ebt, ec, (((1,), (0,)), ((), ())),
                                    preferred_element_type=jnp.float32)
        sqc = sq[:, c * CN:(c + 1) * CN]
        dist = (sqr + sqc) - 2.0 * inner
        for dd in range(CN // 128):
            v = dist[:, dd * 128:(dd + 1) * 128]
            j = lane + (c * CN + dd * 128)
            for l in range(LEVELS):
                lt = v < mv[l]
                nv = jnp.where(lt, v, mv[l])
                ni = jnp.where(lt, j, mi[l])
                v = jnp.where(lt, mv[l], v)
                j = jnp.where(lt, mi[l], j)
                mv[l], mi[l] = nv, ni

    outs = []
    for t in range(K_STATIC):
        win_v = jnp.min(mv[0], axis=1, keepdims=True)
        elig = mv[0] == win_v
        win_j = jnp.min(jnp.where(elig, mi[0], IMAX), axis=1, keepdims=True)
        outs.append(win_j)
        is_win = mi[0] == win_j
        for l in range(LEVELS - 1):
            mv[l] = jnp.where(is_win, mv[l + 1], mv[l])
            mi[l] = jnp.where(is_win, mi[l + 1], mi[l])
        mv[LEVELS - 1] = jnp.where(is_win, INF, mv[LEVELS - 1])
        mi[LEVELS - 1] = jnp.where(is_win, IMAX, mi[LEVELS - 1])
    outs.extend([outs[0], outs[0]])
    idx_out[0] = jnp.concatenate(outs, axis=1) + b * N


def _knn(e8, e8t):
    B, _, N = e8.shape
    return pl.pallas_call(
        _knn_kernel,
        grid=(B, N // TR),
        in_specs=[
            pl.BlockSpec((1, 8, N), lambda b, r: (b, 0, 0)),
            pl.BlockSpec((1, TR, 8), lambda b, r: (b, r, 0)),
        ],
        out_specs=pl.BlockSpec((1, TR, KPAD), lambda b, r: (b, r, 0)),
        out_shape=jax.ShapeDtypeStruct((B, N, KPAD), jnp.int32),
    )(e8, e8t)


def _mlp_kernel(fsem_ref, fins_ref, wsem_ref, bsem_ref, gsem_ref, besem_ref,
                wad_ref, bad_ref, gad_ref, bead_ref,
                wins_ref, bins_ref, gins_ref, beins_ref,
                wie_ref, bie_ref,
                fs_out, fst_out, e8_out, e8t_out):
    s = jnp.sqrt(1.0 + EPS)

    def mlp(w_ref, x, b_ref, g_ref, be_ref):
        y = jnp.dot(_bf(w_ref[...]), _bf(x),
                    preferred_element_type=jnp.float32) + b_ref[...]
        y = y / s * g_ref[...] + be_ref[...]
        return jnp.maximum(y, 0.0)

    fs = mlp(wsem_ref, fsem_ref[0], bsem_ref, gsem_ref, besem_ref)
    adapted = mlp(wad_ref, fs, bad_ref, gad_ref, bead_ref)
    fi = mlp(wins_ref, fins_ref[0], bins_ref, gins_ref, beins_ref)
    f_sins = fi + adapted
    e8 = jnp.dot(_bf(wie_ref[...]), _bf(f_sins),
                 preferred_element_type=jnp.float32) + bie_ref[...]
    fs_out[0] = fs
    fst_out[0] = fs.T
    e8_out[0] = e8
    e8t_out[0] = e8.T


def _mlps(f_sem, f_ins, W_sem, b_sem, g_sem, be_sem, W_ad, b_ad, g_ad, be_ad,
          W_ins, b_ins, g_ins, be_ins, W_ie8, b_ie8):
    B, CI, N = f_sem.shape
    H = W_sem.shape[0]
    TN = 1024
    col = lambda v: v.reshape(-1, 1)
    grid = (B, N // TN)
    wspec = lambda shp: pl.BlockSpec(shp, lambda b, n: (0, 0))
    fs, fst, e8, e8t = pl.pallas_call(
        _mlp_kernel,
        grid=grid,
        in_specs=[
            pl.BlockSpec((1, CI, TN), lambda b, n: (b, 0, n)),
            pl.BlockSpec((1, CI, TN), lambda b, n: (b, 0, n)),
            wspec((H, CI)), wspec((H, 1)), wspec((H, 1)), wspec((H, 1)),
            wspec((H, H)), wspec((H, 1)), wspec((H, 1)), wspec((H, 1)),
            wspec((H, CI)), wspec((H, 1)), wspec((H, 1)), wspec((H, 1)),
            wspec((8, H)), wspec((8, 1)),
        ],
        out_specs=[
            pl.BlockSpec((1, H, TN), lambda b, n: (b, 0, n)),
            pl.BlockSpec((1, TN, H), lambda b, n: (b, n, 0)),
            pl.BlockSpec((1, 8, TN), lambda b, n: (b, 0, n)),
            pl.BlockSpec((1, TN, 8), lambda b, n: (b, n, 0)),
        ],
        out_shape=[
            jax.ShapeDtypeStruct((B, H, N), jnp.float32),
            jax.ShapeDtypeStruct((B, N, H), jnp.float32),
            jax.ShapeDtypeStruct((B, 8, N), jnp.float32),
            jax.ShapeDtypeStruct((B, N, 8), jnp.float32),
        ],
    )(f_sem, f_ins,
      W_sem, col(b_sem), col(g_sem), col(be_sem),
      W_ad, col(b_ad), col(g_ad), col(be_ad),
      W_ins, col(b_ins), col(g_ins), col(be_ins),
      W_ie8, col(b_ie8))
    return fs, fst, e8, e8t


def kernel(f_sem, f_ins, W_sem, b_sem, g_sem, be_sem, W_ad, b_ad, g_ad, be_ad,
           W_ins, b_ins, g_ins, be_ins, W_sp, b_sp, W_ie, b_ie, k):
    C_ie = W_ie.shape[0]
    W_ie8 = jnp.zeros((8, W_ie.shape[1]), jnp.float32).at[:C_ie].set(W_ie)
    b_ie8 = jnp.zeros((8,), jnp.float32).at[:C_ie].set(b_ie)

    fs, fst, e8, e8t = _mlps(f_sem, f_ins, W_sem, b_sem, g_sem, be_sem,
                             W_ad, b_ad, g_ad, be_ad,
                             W_ins, b_ins, g_ins, be_ins, W_ie8, b_ie8)
    B, H, N = fs.shape
    e_ins = e8[:, :C_ie, :]

    idx32 = _knn(e8, e8t)

    fst_flat = fst.reshape(B * N, H)
    gathered = fst_flat[idx32[:, :, :K_STATIC].reshape(-1)]
    f_isem = jnp.max(gathered.reshape(B, N, K_STATIC, H), axis=2)
    f_isem = f_isem.transpose(0, 2, 1)
    p_sem = jnp.einsum('oi,bin->bon', _bf(W_sp), _bf(f_isem),
                       preferred_element_type=jnp.float32) + b_sp[None, :, None]
    return (p_sem, e_ins)


# speedup vs baseline: 10.9538x; 3.2176x over previous
import functools
import jax
import jax.numpy as jnp
from jax import lax
from jax.experimental import pallas as pl
from jax.experimental.pallas import tpu as pltpu
from jax.experimental.pallas import tpu_sc as plsc

EPS = 1e-5
K_STATIC = 30
KPAD = 32
LEVELS = 6
TR = 64
GR = 8
CN = 1024
INF = float('inf')
IMAX = 2**31 - 1
JBIG = 1.0e9


def _bf(x):
    return x.astype(jnp.bfloat16)


def _knn_kernel(e8_ref, e8t_ref, idx_out):
    b = pl.program_id(0)
    e8 = e8_ref[0]
    N = e8.shape[1]
    eb = e8.astype(jnp.bfloat16)
    sq = e8[0:1] * e8[0:1]
    for c in range(1, 5):
        sq = sq + e8[c:c+1] * e8[c:c+1]
    et = e8t_ref[0]
    ebt = et.astype(jnp.bfloat16)
    sqr = et[:, 0:1] * et[:, 0:1]
    for c in range(1, 5):
        sqr = sqr + et[:, c:c+1] * et[:, c:c+1]

    mv = [jnp.full((TR, 128), INF, jnp.float32) for _ in range(LEVELS)]
    mi = [jnp.full((TR, 128), JBIG, jnp.float32) for _ in range(LEVELS)]
    lane = jax.lax.broadcasted_iota(jnp.int32, (TR, 128), 1).astype(jnp.float32)

    for c in range(N // CN):
        ec = eb[:, c * CN:(c + 1) * CN]
        inner = jax.lax.dot_general(ebt, ec, (((1,), (0,)), ((), ())),
                                    preferred_element_type=jnp.float32)
        sqc = sq[:, c * CN:(c + 1) * CN]
        dist = (sqr + sqc) - 2.0 * inner
        for dd in range(CN // 128):
            v = dist[:, dd * 128:(dd + 1) * 128]
            j = lane + (c * CN + dd * 128)
            for l in range(LEVELS):
                lt = v < mv[l]
                nv = jnp.where(lt, v, mv[l])
                ni = jnp.where(lt, j, mi[l])
                v = jnp.where(lt, mv[l], v)
                j = jnp.where(lt, mi[l], j)
                mv[l], mi[l] = nv, ni

    NG = TR // GR
    gv = [[mv[l][g * GR:(g + 1) * GR] for l in range(LEVELS)]
          for g in range(NG)]
    gi = [[mi[l][g * GR:(g + 1) * GR] for l in range(LEVELS)]
          for g in range(NG)]
    outs = [[] for _ in range(NG)]
    for t in range(K_STATIC):
        for g in range(NG):
            win_v = jnp.min(gv[g][0], axis=1, keepdims=True)
            elig = gv[g][0] == win_v
            win_j = jnp.min(jnp.where(elig, gi[g][0], JBIG), axis=1,
                            keepdims=True)
            outs[g].append(win_j)
            is_win = gi[g][0] == win_j
            for l in range(LEVELS - 1):
                gv[g][l] = jnp.where(is_win, gv[g][l + 1], gv[g][l])
                gi[g][l] = jnp.where(is_win, gi[g][l + 1], gi[g][l])
            gv[g][LEVELS - 1] = jnp.where(is_win, INF, gv[g][LEVELS - 1])
            gi[g][LEVELS - 1] = jnp.where(is_win, JBIG, gi[g][LEVELS - 1])
    group_res = [jnp.concatenate(outs[g] + [outs[g][0], outs[g][0]], axis=1)
                 for g in range(NG)]
    idxf = jnp.concatenate(group_res, axis=0)
    idx_out[0] = idxf.astype(jnp.int32) + b * N


def _knn(e8, e8t):
    B, _, N = e8.shape
    return pl.pallas_call(
        _knn_kernel,
        grid=(B, N // TR),
        in_specs=[
            pl.BlockSpec((1, 8, N), lambda b, r: (b, 0, 0)),
            pl.BlockSpec((1, TR, 8), lambda b, r: (b, r, 0)),
        ],
        out_specs=pl.BlockSpec((1, TR, KPAD), lambda b, r: (b, r, 0)),
        out_shape=jax.ShapeDtypeStruct((B, N, KPAD), jnp.int32),
    )(e8, e8t)


def _sc_gather_max(NPTS, H):
    info = plsc.get_sparse_core_info()
    NC, NS, L = info.num_cores, info.num_subcores, info.num_lanes
    NW = NC * NS
    pts_per_w = NPTS // NW
    P = 8
    nchunk = pts_per_w // P
    NCH = H // L
    mesh = plsc.VectorSubcoreMesh(core_axis_name="c", subcore_axis_name="s")

    @functools.partial(
        pl.kernel, mesh=mesh,
        out_type=jax.ShapeDtypeStruct((NPTS, H), jnp.float32),
        scratch_types=[
            pltpu.VMEM((P * KPAD,), jnp.int32),
            pltpu.VMEM((P * KPAD, H), jnp.float32),
            pltpu.VMEM((P, H), jnp.float32),
            pltpu.SemaphoreType.DMA,
        ],
    )
    def sc_kernel(tab_hbm, idx_hbm, out_hbm, idx_v, rows_v, acc_v, sem):
        wid = lax.axis_index("s") * NC + lax.axis_index("c")
        base = wid * pts_per_w

        def chunk_body(ci, carry):
            p0 = base + ci * P
            pltpu.sync_copy(idx_hbm.at[pl.ds(p0 * KPAD, P * KPAD)], idx_v)
            pltpu.async_copy(tab_hbm.at[idx_v], rows_v, sem).wait()
            for p in range(P):
                accs = [rows_v[p * KPAD, pl.ds(ch * L, L)]
                        for ch in range(NCH)]

                def k_body(kk, accs):
                    return tuple(
                        jnp.maximum(a, rows_v[p * KPAD + kk,
                                              pl.ds(ch * L, L)])
                        for ch, a in enumerate(accs))

                accs = lax.fori_loop(1, KPAD, k_body, tuple(accs))
                for ch in range(NCH):
                    acc_v[p, pl.ds(ch * L, L)] = accs[ch]
            pltpu.sync_copy(acc_v, out_hbm.at[pl.ds(p0, P)])
            return carry

        lax.fori_loop(0, nchunk, chunk_body, 0)

    return sc_kernel


def _psem_kernel(x_ref, w_ref, b_ref, out_ref):
    xt = x_ref[0].T
    out_ref[0] = (jnp.dot(_bf(w_ref[...]), _bf(xt),
                          preferred_element_type=jnp.float32) + b_ref[...])


def _psem(f_isemT, W_sp16, b_sp16):
    B_N, H = f_isemT.shape
    TN4 = 2048
    x = f_isemT.reshape(-1, TN4, H)
    G = x.shape[0]
    out = pl.pallas_call(
        _psem_kernel,
        grid=(G,),
        in_specs=[
            pl.BlockSpec((1, TN4, H), lambda g: (g, 0, 0)),
            pl.BlockSpec((16, H), lambda g: (0, 0)),
            pl.BlockSpec((16, 1), lambda g: (0, 0)),
        ],
        out_specs=pl.BlockSpec((1, 16, TN4), lambda g: (g, 0, 0)),
        out_shape=jax.ShapeDtypeStruct((G, 16, TN4), jnp.float32),
    )(x, W_sp16, b_sp16)
    return out


def _mlp_kernel(fsem_ref, fins_ref, wsem_ref, bsem_ref, gsem_ref, besem_ref,
                wad_ref, bad_ref, gad_ref, bead_ref,
                wins_ref, bins_ref, gins_ref, beins_ref,
                wie_ref, bie_ref,
                fs_out, fst_out, e8_out, e8t_out):
    s = jnp.sqrt(1.0 + EPS)

    def mlp(w_ref, x, b_ref, g_ref, be_ref):
        y = jnp.dot(_bf(w_ref[...]), _bf(x),
                    preferred_element_type=jnp.float32) + b_ref[...]
        y = y / s * g_ref[...] + be_ref[...]
        return jnp.maximum(y, 0.0)

    fs = mlp(wsem_ref, fsem_ref[0], bsem_ref, gsem_ref, besem_ref)
    adapted = mlp(wad_ref, fs, bad_ref, gad_ref, bead_ref)
    fi = mlp(wins_ref, fins_ref[0], bins_ref, gins_ref, beins_ref)
    f_sins = fi + adapted
    e8 = jnp.dot(_bf(wie_ref[...]), _bf(f_sins),
                 preferred_element_type=jnp.float32) + bie_ref[...]
    fs_out[0] = fs
    fst_out[0] = fs.T
    e8_out[0] = e8
    e8t_out[0] = e8.T


def _mlps(f_sem, f_ins, W_sem, b_sem, g_sem, be_sem, W_ad, b_ad, g_ad, be_ad,
          W_ins, b_ins, g_ins, be_ins, W_ie8, b_ie8):
    B, CI, N = f_sem.shape
    H = W_sem.shape[0]
    TN = 1024
    col = lambda v: v.reshape(-1, 1)
    grid = (B, N // TN)
    wspec = lambda shp: pl.BlockSpec(shp, lambda b, n: (0, 0))
    fs, fst, e8, e8t = pl.pallas_call(
        _mlp_kernel,
        grid=grid,
        in_specs=[
            pl.BlockSpec((1, CI, TN), lambda b, n: (b, 0, n)),
            pl.BlockSpec((1, CI, TN), lambda b, n: (b, 0, n)),
            wspec((H, CI)), wspec((H, 1)), wspec((H, 1)), wspec((H, 1)),
            wspec((H, H)), wspec((H, 1)), wspec((H, 1)), wspec((H, 1)),
            wspec((H, CI)), wspec((H, 1)), wspec((H, 1)), wspec((H, 1)),
            wspec((8, H)), wspec((8, 1)),
        ],
        out_specs=[
            pl.BlockSpec((1, H, TN), lambda b, n: (b, 0, n)),
            pl.BlockSpec((1, TN, H), lambda b, n: (b, n, 0)),
            pl.BlockSpec((1, 8, TN), lambda b, n: (b, 0, n)),
            pl.BlockSpec((1, TN, 8), lambda b, n: (b, n, 0)),
        ],
        out_shape=[
            jax.ShapeDtypeStruct((B, H, N), jnp.float32),
            jax.ShapeDtypeStruct((B, N, H), jnp.float32),
            jax.ShapeDtypeStruct((B, 8, N), jnp.float32),
            jax.ShapeDtypeStruct((B, N, 8), jnp.float32),
        ],
    )(f_sem, f_ins,
      W_sem, col(b_sem), col(g_sem), col(be_sem),
      W_ad, col(b_ad), col(g_ad), col(be_ad),
      W_ins, col(b_ins), col(g_ins), col(be_ins),
      W_ie8, col(b_ie8))
    return fs, fst, e8, e8t


def kernel(f_sem, f_ins, W_sem, b_sem, g_sem, be_sem, W_ad, b_ad, g_ad, be_ad,
           W_ins, b_ins, g_ins, be_ins, W_sp, b_sp, W_ie, b_ie, k):
    C_ie = W_ie.shape[0]
    W_ie8 = jnp.zeros((8, W_ie.shape[1]), jnp.float32).at[:C_ie].set(W_ie)
    b_ie8 = jnp.zeros((8,), jnp.float32).at[:C_ie].set(b_ie)

    fs, fst, e8, e8t = _mlps(f_sem, f_ins, W_sem, b_sem, g_sem, be_sem,
                             W_ad, b_ad, g_ad, be_ad,
                             W_ins, b_ins, g_ins, be_ins, W_ie8, b_ie8)
    B, H, N = fs.shape
    e_ins = e8[:, :C_ie, :]

    idx32 = _knn(e8, e8t)

    fst_flat = fst.reshape(B * N, H)
    idx_flat = idx32.reshape(B * N * KPAD)
    f_isemT = _sc_gather_max(B * N, H)(fst_flat, idx_flat)

    C_sp = W_sp.shape[0]
    W_sp16 = jnp.zeros((16, H), jnp.float32).at[:C_sp].set(W_sp)
    b_sp16 = jnp.zeros((16,), jnp.float32).at[:C_sp].set(b_sp).reshape(-1, 1)
    out = _psem(f_isemT, W_sp16, b_sp16)
    p_sem = out.reshape(B, -1, 16, out.shape[-1]).transpose(0, 2, 1, 3)
    p_sem = p_sem.reshape(B, 16, N)[:, :C_sp, :]
    return (p_sem, e_ins)


# speedup vs baseline: 11.6905x; 1.0673x over previous
import functools
import jax
import jax.numpy as jnp
from jax import lax
from jax.experimental import pallas as pl
from jax.experimental.pallas import tpu as pltpu
from jax.experimental.pallas import tpu_sc as plsc

EPS = 1e-5
K_STATIC = 30
KPAD = 32
LEVELS = 6
TR = 64
GR = 8
CN = 1024
INF = float('inf')
IMAX = 2**31 - 1
JBIG = 1.0e9


def _bf(x):
    return x.astype(jnp.bfloat16)


def _knn_kernel(e8_ref, e8t_ref, idx_out):
    b = pl.program_id(0)
    e8 = e8_ref[0]
    N = e8.shape[1]
    eb = e8.astype(jnp.bfloat16)
    sq = e8[0:1] * e8[0:1]
    for c in range(1, 5):
        sq = sq + e8[c:c+1] * e8[c:c+1]
    et = e8t_ref[0]
    ebt = et.astype(jnp.bfloat16)
    sqr = et[:, 0:1] * et[:, 0:1]
    for c in range(1, 5):
        sqr = sqr + et[:, c:c+1] * et[:, c:c+1]

    mv = [jnp.full((TR, 128), INF, jnp.float32) for _ in range(LEVELS)]
    mi = [jnp.full((TR, 128), JBIG, jnp.float32) for _ in range(LEVELS)]
    lane = jax.lax.broadcasted_iota(jnp.int32, (TR, 128), 1).astype(jnp.float32)

    for c in range(N // CN):
        ec = eb[:, c * CN:(c + 1) * CN]
        inner = jax.lax.dot_general(ebt, ec, (((1,), (0,)), ((), ())),
                                    preferred_element_type=jnp.float32)
        sqc = sq[:, c * CN:(c + 1) * CN]
        dist = (sqr + sqc) - 2.0 * inner
        for dd in range(CN // 128):
            v = dist[:, dd * 128:(dd + 1) * 128]
            j = lane + (c * CN + dd * 128)
            for l in range(LEVELS):
                lt = v < mv[l]
                nv = jnp.where(lt, v, mv[l])
                ni = jnp.where(lt, j, mi[l])
                v = jnp.where(lt, mv[l], v)
                j = jnp.where(lt, mi[l], j)
                mv[l], mi[l] = nv, ni

    NG = TR // GR
    gv = [[mv[l][g * GR:(g + 1) * GR] for l in range(LEVELS)]
          for g in range(NG)]
    gi = [[mi[l][g * GR:(g + 1) * GR] for l in range(LEVELS)]
          for g in range(NG)]
    outs = [[] for _ in range(NG)]
    for t in range(K_STATIC):
        for g in range(NG):
            win_v = jnp.min(gv[g][0], axis=1, keepdims=True)
            elig = gv[g][0] == win_v
            win_j = jnp.min(jnp.where(elig, gi[g][0], JBIG), axis=1,
                            keepdims=True)
            outs[g].append(win_j)
            is_win = gi[g][0] == win_j
            for l in range(LEVELS - 1):
                gv[g][l] = jnp.where(is_win, gv[g][l + 1], gv[g][l])
                gi[g][l] = jnp.where(is_win, gi[g][l + 1], gi[g][l])
            gv[g][LEVELS - 1] = jnp.where(is_win, INF, gv[g][LEVELS - 1])
            gi[g][LEVELS - 1] = jnp.where(is_win, JBIG, gi[g][LEVELS - 1])
    group_res = [jnp.concatenate(outs[g] + [outs[g][0], outs[g][0]], axis=1)
                 for g in range(NG)]
    idxf = jnp.concatenate(group_res, axis=0)
    idx_out[0] = idxf.astype(jnp.int32) + b * N


def _knn(e8, e8t):
    B, _, N = e8.shape
    return pl.pallas_call(
        _knn_kernel,
        grid=(B, N // TR),
        in_specs=[
            pl.BlockSpec((1, 8, N), lambda b, r: (b, 0, 0)),
            pl.BlockSpec((1, TR, 8), lambda b, r: (b, r, 0)),
        ],
        out_specs=pl.BlockSpec((1, TR, KPAD), lambda b, r: (b, r, 0)),
        out_shape=jax.ShapeDtypeStruct((B, N, KPAD), jnp.int32),
    )(e8, e8t)


def _sc_gather_max(NPTS, H):
    info = plsc.get_sparse_core_info()
    NC, NS, L = info.num_cores, info.num_subcores, info.num_lanes
    NW = NC * NS
    pts_per_w = NPTS // NW
    P = 8
    nchunk = pts_per_w // P
    NCH = H // L
    mesh = plsc.VectorSubcoreMesh(core_axis_name="c", subcore_axis_name="s")

    @functools.partial(
        pl.kernel, mesh=mesh,
        out_type=jax.ShapeDtypeStruct((NPTS, H), jnp.float32),
        scratch_types=[
            pltpu.VMEM((P * KPAD,), jnp.int32),
            pltpu.VMEM((P * KPAD,), jnp.int32),
            pltpu.VMEM((2, P * KPAD, H), jnp.float32),
            pltpu.VMEM((2, P, H), jnp.float32),
            pltpu.SemaphoreType.DMA,
            pltpu.SemaphoreType.DMA,
            pltpu.SemaphoreType.DMA,
            pltpu.SemaphoreType.DMA,
        ],
    )
    def sc_kernel(tab_hbm, idx_hbm, out_hbm, idx_c0, idx_c1, rows, acc,
                  gsem0, gsem1, osem0, osem1):
        wid = lax.axis_index("s") * NC + lax.axis_index("c")
        base = wid * pts_per_w
        gsems = (gsem0, gsem1)
        osems = (osem0, osem1)
        idx_cs = (idx_c0, idx_c1)

        def start_gather(ci, par):
            @pl.when(ci < nchunk)
            def _():
                p0 = base + ci * P
                pltpu.sync_copy(idx_hbm.at[pl.ds(p0 * KPAD, P * KPAD)],
                                idx_cs[par])
                pltpu.async_copy(tab_hbm.at[idx_cs[par]],
                                 rows.at[par], gsems[par])

        start_gather(0, 0)

        def compute(cj, ci, par):
            rv = rows.at[par]
            av = acc.at[par]
            p0 = base + ci * P
            pltpu.make_async_copy(tab_hbm.at[idx_cs[par]], rv,
                                  gsems[par]).wait()

            @pl.when(cj > 0)
            def _():
                pltpu.make_async_copy(av, out_hbm.at[pl.ds(p0, P)],
                                      osems[par]).wait()

            for p in range(P):
                accs = [rv[p * KPAD, pl.ds(ch * L, L)] for ch in range(NCH)]

                def k_body(kk, accs):
                    return tuple(
                        jnp.maximum(a, rv[p * KPAD + kk, pl.ds(ch * L, L)])
                        for ch, a in enumerate(accs))

                accs = lax.fori_loop(1, KPAD, k_body, tuple(accs))
                for ch in range(NCH):
                    av[p, pl.ds(ch * L, L)] = accs[ch]
            pltpu.async_copy(av, out_hbm.at[pl.ds(p0, P)], osems[par])

        def body2(cj, carry):
            ci0 = cj * 2
            start_gather(ci0 + 1, 1)
            compute(cj, ci0, 0)
            start_gather(ci0 + 2, 0)
            compute(cj, ci0 + 1, 1)
            return carry

        lax.fori_loop(0, nchunk // 2, body2, 0)
        pltpu.make_async_copy(acc.at[0], out_hbm.at[pl.ds(base, P)],
                              osem0).wait()
        pltpu.make_async_copy(acc.at[1], out_hbm.at[pl.ds(base, P)],
                              osem1).wait()

    return sc_kernel


def _psem_kernel(x_ref, w_ref, b_ref, out_ref):
    xt = x_ref[0].T
    out_ref[0] = (jnp.dot(_bf(w_ref[...]), _bf(xt),
                          preferred_element_type=jnp.float32) + b_ref[...])


def _psem(f_isemT, W_sp16, b_sp16):
    B_N, H = f_isemT.shape
    TN4 = 2048
    x = f_isemT.reshape(-1, TN4, H)
    G = x.shape[0]
    out = pl.pallas_call(
        _psem_kernel,
        grid=(G,),
        in_specs=[
            pl.BlockSpec((1, TN4, H), lambda g: (g, 0, 0)),
            pl.BlockSpec((16, H), lambda g: (0, 0)),
            pl.BlockSpec((16, 1), lambda g: (0, 0)),
        ],
        out_specs=pl.BlockSpec((1, 16, TN4), lambda g: (g, 0, 0)),
        out_shape=jax.ShapeDtypeStruct((G, 16, TN4), jnp.float32),
    )(x, W_sp16, b_sp16)
    return out


def _mlp_kernel(fsem_ref, fins_ref, wsem_ref, bsem_ref, gsem_ref, besem_ref,
                wad_ref, bad_ref, gad_ref, bead_ref,
                wins_ref, bins_ref, gins_ref, beins_ref,
                wie_ref, bie_ref,
                fs_out, fst_out, e8_out, e8t_out):
    s = jnp.sqrt(1.0 + EPS)

    def mlp(w_ref, x, b_ref, g_ref, be_ref):
        y = jnp.dot(_bf(w_ref[...]), _bf(x),
                    preferred_element_type=jnp.float32) + b_ref[...]
        y = y / s * g_ref[...] + be_ref[...]
        return jnp.maximum(y, 0.0)

    fs = mlp(wsem_ref, fsem_ref[0], bsem_ref, gsem_ref, besem_ref)
    adapted = mlp(wad_ref, fs, bad_ref, gad_ref, bead_ref)
    fi = mlp(wins_ref, fins_ref[0], bins_ref, gins_ref, beins_ref)
    f_sins = fi + adapted
    e8 = jnp.dot(_bf(wie_ref[...]), _bf(f_sins),
                 preferred_element_type=jnp.float32) + bie_ref[...]
    fs_out[0] = fs
    fst_out[0] = fs.T
    e8_out[0] = e8
    e8t_out[0] = e8.T


def _mlps(f_sem, f_ins, W_sem, b_sem, g_sem, be_sem, W_ad, b_ad, g_ad, be_ad,
          W_ins, b_ins, g_ins, be_ins, W_ie8, b_ie8):
    B, CI, N = f_sem.shape
    H = W_sem.shape[0]
    TN = 1024
    col = lambda v: v.reshape(-1, 1)
    grid = (B, N // TN)
    wspec = lambda shp: pl.BlockSpec(shp, lambda b, n: (0, 0))
    fs, fst, e8, e8t = pl.pallas_call(
        _mlp_kernel,
        grid=grid,
        in_specs=[
            pl.BlockSpec((1, CI, TN), lambda b, n: (b, 0, n)),
            pl.BlockSpec((1, CI, TN), lambda b, n: (b, 0, n)),
            wspec((H, CI)), wspec((H, 1)), wspec((H, 1)), wspec((H, 1)),
            wspec((H, H)), wspec((H, 1)), wspec((H, 1)), wspec((H, 1)),
            wspec((H, CI)), wspec((H, 1)), wspec((H, 1)), wspec((H, 1)),
            wspec((8, H)), wspec((8, 1)),
        ],
        out_specs=[
            pl.BlockSpec((1, H, TN), lambda b, n: (b, 0, n)),
            pl.BlockSpec((1, TN, H), lambda b, n: (b, n, 0)),
            pl.BlockSpec((1, 8, TN), lambda b, n: (b, 0, n)),
            pl.BlockSpec((1, TN, 8), lambda b, n: (b, n, 0)),
        ],
        out_shape=[
            jax.ShapeDtypeStruct((B, H, N), jnp.float32),
            jax.ShapeDtypeStruct((B, N, H), jnp.float32),
            jax.ShapeDtypeStruct((B, 8, N), jnp.float32),
            jax.ShapeDtypeStruct((B, N, 8), jnp.float32),
        ],
    )(f_sem, f_ins,
      W_sem, col(b_sem), col(g_sem), col(be_sem),
      W_ad, col(b_ad), col(g_ad), col(be_ad),
      W_ins, col(b_ins), col(g_ins), col(be_ins),
      W_ie8, col(b_ie8))
    return fs, fst, e8, e8t


def kernel(f_sem, f_ins, W_sem, b_sem, g_sem, be_sem, W_ad, b_ad, g_ad, be_ad,
           W_ins, b_ins, g_ins, be_ins, W_sp, b_sp, W_ie, b_ie, k):
    C_ie = W_ie.shape[0]
    W_ie8 = jnp.zeros((8, W_ie.shape[1]), jnp.float32).at[:C_ie].set(W_ie)
    b_ie8 = jnp.zeros((8,), jnp.float32).at[:C_ie].set(b_ie)

    fs, fst, e8, e8t = _mlps(f_sem, f_ins, W_sem, b_sem, g_sem, be_sem,
                             W_ad, b_ad, g_ad, be_ad,
                             W_ins, b_ins, g_ins, be_ins, W_ie8, b_ie8)
    B, H, N = fs.shape
    e_ins = e8[:, :C_ie, :]

    idx32 = _knn(e8, e8t)

    fst_flat = fst.reshape(B * N, H)
    idx_flat = idx32.reshape(B * N * KPAD)
    f_isemT = _sc_gather_max(B * N, H)(fst_flat, idx_flat)

    C_sp = W_sp.shape[0]
    W_sp16 = jnp.zeros((16, H), jnp.float32).at[:C_sp].set(W_sp)
    b_sp16 = jnp.zeros((16,), jnp.float32).at[:C_sp].set(b_sp).reshape(-1, 1)
    out = _psem(f_isemT, W_sp16, b_sp16)
    p_sem = out.reshape(B, -1, 16, out.shape[-1]).transpose(0, 2, 1, 3)
    p_sem = p_sem.reshape(B, 16, N)[:, :C_sp, :]
    return (p_sem, e_ins)


# speedup vs baseline: 17.6845x; 1.5127x over previous
import functools
import jax
import jax.numpy as jnp
from jax import lax
from jax.experimental import pallas as pl
from jax.experimental.pallas import tpu as pltpu
from jax.experimental.pallas import tpu_sc as plsc

EPS = 1e-5
K_STATIC = 30
KPAD = 32
LEVELS = 6
TR = 128
GR = 8
CN = 1024
INF = float('inf')
IMAX = 2**31 - 1
JBIG = 1.0e9


def _bf(x):
    return x.astype(jnp.bfloat16)


def _knn_kernel(e8_ref, e8t_ref, idx_out):
    b = pl.program_id(0)
    e8 = e8_ref[0]
    N = e8.shape[1]
    eb = e8.astype(jnp.bfloat16)
    sq = e8[0:1] * e8[0:1]
    for c in range(1, 5):
        sq = sq + e8[c:c+1] * e8[c:c+1]
    et = e8t_ref[0]
    ebt = et.astype(jnp.bfloat16)
    sqr = et[:, 0:1] * et[:, 0:1]
    for c in range(1, 5):
        sqr = sqr + et[:, c:c+1] * et[:, c:c+1]

    mv = [jnp.full((TR, 128), INF, jnp.float32) for _ in range(LEVELS)]
    mi = [jnp.full((TR, 128), JBIG, jnp.float32) for _ in range(LEVELS)]
    lane = jax.lax.broadcasted_iota(jnp.int32, (TR, 128), 1).astype(jnp.float32)

    for c in range(N // CN):
        ec = eb[:, c * CN:(c + 1) * CN]
        inner = jax.lax.dot_general(ebt, ec, (((1,), (0,)), ((), ())),
                                    preferred_element_type=jnp.float32)
        sqc = sq[:, c * CN:(c + 1) * CN]
        dist = (sqr + sqc) - 2.0 * inner
        for dd in range(CN // 128):
            v = dist[:, dd * 128:(dd + 1) * 128]
            j = lane + (c * CN + dd * 128)
            for l in range(LEVELS):
                lt = v < mv[l]
                nv = jnp.where(lt, v, mv[l])
                ni = jnp.where(lt, j, mi[l])
                v = jnp.where(lt, mv[l], v)
                j = jnp.where(lt, mi[l], j)
                mv[l], mi[l] = nv, ni

    NG = TR // GR
    gv = [[mv[l][g * GR:(g + 1) * GR] for l in range(LEVELS)]
          for g in range(NG)]
    gi = [[mi[l][g * GR:(g + 1) * GR] for l in range(LEVELS)]
          for g in range(NG)]
    outs = [[] for _ in range(NG)]
    for t in range(K_STATIC):
        for g in range(NG):
            win_v = jnp.min(gv[g][0], axis=1, keepdims=True)
            elig = gv[g][0] == win_v
            win_j = jnp.min(jnp.where(elig, gi[g][0], JBIG), axis=1,
                            keepdims=True)
            outs[g].append(win_j)
            is_win = gi[g][0] == win_j
            for l in range(LEVELS - 1):
                gv[g][l] = jnp.where(is_win, gv[g][l + 1], gv[g][l])
                gi[g][l] = jnp.where(is_win, gi[g][l + 1], gi[g][l])
            gv[g][LEVELS - 1] = jnp.where(is_win, INF, gv[g][LEVELS - 1])
            gi[g][LEVELS - 1] = jnp.where(is_win, JBIG, gi[g][LEVELS - 1])
    group_res = [jnp.concatenate(outs[g] + [outs[g][0], outs[g][0]], axis=1)
                 for g in range(NG)]
    idxf = jnp.concatenate(group_res, axis=0)
    idx_out[0] = idxf.astype(jnp.int32) + b * N


def _knn(e8, e8t):
    B, _, N = e8.shape
    return pl.pallas_call(
        _knn_kernel,
        grid=(B, N // TR),
        in_specs=[
            pl.BlockSpec((1, 8, N), lambda b, r: (b, 0, 0)),
            pl.BlockSpec((1, TR, 8), lambda b, r: (b, r, 0)),
        ],
        out_specs=pl.BlockSpec((1, TR, KPAD), lambda b, r: (b, r, 0)),
        out_shape=jax.ShapeDtypeStruct((B, N, KPAD), jnp.int32),
    )(e8, e8t)


def _sc_gather_max(NPTS, H):
    info = plsc.get_sparse_core_info()
    NC, NS, L = info.num_cores, info.num_subcores, info.num_lanes
    NW = NC * NS
    pts_per_w = NPTS // NW
    P = 8
    nchunk = pts_per_w // P
    NCH = H // L
    mesh = plsc.VectorSubcoreMesh(core_axis_name="c", subcore_axis_name="s")

    @functools.partial(
        pl.kernel, mesh=mesh,
        out_type=jax.ShapeDtypeStruct((NPTS, H), jnp.float32),
        scratch_types=[
            pltpu.VMEM((P * KPAD,), jnp.int32),
            pltpu.VMEM((P * KPAD,), jnp.int32),
            pltpu.VMEM((2, P * KPAD, H), jnp.float32),
            pltpu.VMEM((2, P, H), jnp.float32),
            pltpu.SemaphoreType.DMA,
            pltpu.SemaphoreType.DMA,
            pltpu.SemaphoreType.DMA,
            pltpu.SemaphoreType.DMA,
        ],
    )
    def sc_kernel(tab_hbm, idx_hbm, out_hbm, idx_c0, idx_c1, rows, acc,
                  gsem0, gsem1, osem0, osem1):
        wid = lax.axis_index("s") * NC + lax.axis_index("c")
        base = wid * pts_per_w
        gsems = (gsem0, gsem1)
        osems = (osem0, osem1)
        idx_cs = (idx_c0, idx_c1)

        def start_gather(ci, par):
            @pl.when(ci < nchunk)
            def _():
                p0 = base + ci * P
                pltpu.sync_copy(idx_hbm.at[pl.ds(p0 * KPAD, P * KPAD)],
                                idx_cs[par])
                pltpu.async_copy(tab_hbm.at[idx_cs[par]],
                                 rows.at[par], gsems[par])

        start_gather(0, 0)

        def compute(cj, ci, par):
            rv = rows.at[par]
            av = acc.at[par]
            p0 = base + ci * P
            pltpu.make_async_copy(tab_hbm.at[idx_cs[par]], rv,
                                  gsems[par]).wait()

            @pl.when(cj > 0)
            def _():
                pltpu.make_async_copy(av, out_hbm.at[pl.ds(p0, P)],
                                      osems[par]).wait()

            for p in range(P):
                accs = [rv[p * KPAD, pl.ds(ch * L, L)] for ch in range(NCH)]

                def k_body(kk, accs):
                    return tuple(
                        jnp.maximum(a, rv[p * KPAD + kk, pl.ds(ch * L, L)])
                        for ch, a in enumerate(accs))

                accs = lax.fori_loop(1, KPAD, k_body, tuple(accs))
                for ch in range(NCH):
                    av[p, pl.ds(ch * L, L)] = accs[ch]
            pltpu.async_copy(av, out_hbm.at[pl.ds(p0, P)], osems[par])

        def body2(cj, carry):
            ci0 = cj * 2
            start_gather(ci0 + 1, 1)
            compute(cj, ci0, 0)
            start_gather(ci0 + 2, 0)
            compute(cj, ci0 + 1, 1)
            return carry

        lax.fori_loop(0, nchunk // 2, body2, 0)
        pltpu.make_async_copy(acc.at[0], out_hbm.at[pl.ds(base, P)],
                              osem0).wait()
        pltpu.make_async_copy(acc.at[1], out_hbm.at[pl.ds(base, P)],
                              osem1).wait()

    return sc_kernel


def _psem_kernel(x_ref, w_ref, b_ref, out_ref):
    xt = x_ref[0].T
    out_ref[0] = (jnp.dot(_bf(w_ref[...]), _bf(xt),
                          preferred_element_type=jnp.float32) + b_ref[...])


def _psem(f_isemT, W_sp16, b_sp16):
    B_N, H = f_isemT.shape
    TN4 = 2048
    x = f_isemT.reshape(-1, TN4, H)
    G = x.shape[0]
    out = pl.pallas_call(
        _psem_kernel,
        grid=(G,),
        in_specs=[
            pl.BlockSpec((1, TN4, H), lambda g: (g, 0, 0)),
            pl.BlockSpec((16, H), lambda g: (0, 0)),
            pl.BlockSpec((16, 1), lambda g: (0, 0)),
        ],
        out_specs=pl.BlockSpec((1, 16, TN4), lambda g: (g, 0, 0)),
        out_shape=jax.ShapeDtypeStruct((G, 16, TN4), jnp.float32),
    )(x, W_sp16, b_sp16)
    return out


def _mlp_kernel(fsem_ref, fins_ref, wsem_ref, bsem_ref, gsem_ref, besem_ref,
                wad_ref, bad_ref, gad_ref, bead_ref,
                wins_ref, bins_ref, gins_ref, beins_ref,
                wie_ref, bie_ref,
                fs_out, fst_out, e8_out, e8t_out):
    s = jnp.sqrt(1.0 + EPS)

    def mlp(w_ref, x, b_ref, g_ref, be_ref):
        y = jnp.dot(_bf(w_ref[...]), _bf(x),
                    preferred_element_type=jnp.float32) + b_ref[...]
        y = y / s * g_ref[...] + be_ref[...]
        return jnp.maximum(y, 0.0)

    fs = mlp(wsem_ref, fsem_ref[0], bsem_ref, gsem_ref, besem_ref)
    adapted = mlp(wad_ref, fs, bad_ref, gad_ref, bead_ref)
    fi = mlp(wins_ref, fins_ref[0], bins_ref, gins_ref, beins_ref)
    f_sins = fi + adapted
    e8 = jnp.dot(_bf(wie_ref[...]), _bf(f_sins),
                 preferred_element_type=jnp.float32) + bie_ref[...]
    fs_out[0] = fs
    fst_out[0] = fs.T
    e8_out[0] = e8
    e8t_out[0] = e8.T


def _mlps(f_sem, f_ins, W_sem, b_sem, g_sem, be_sem, W_ad, b_ad, g_ad, be_ad,
          W_ins, b_ins, g_ins, be_ins, W_ie8, b_ie8):
    B, CI, N = f_sem.shape
    H = W_sem.shape[0]
    TN = 1024
    col = lambda v: v.reshape(-1, 1)
    grid = (B, N // TN)
    wspec = lambda shp: pl.BlockSpec(shp, lambda b, n: (0, 0))
    fs, fst, e8, e8t = pl.pallas_call(
        _mlp_kernel,
        grid=grid,
        in_specs=[
            pl.BlockSpec((1, CI, TN), lambda b, n: (b, 0, n)),
            pl.BlockSpec((1, CI, TN), lambda b, n: (b, 0, n)),
            wspec((H, CI)), wspec((H, 1)), wspec((H, 1)), wspec((H, 1)),
            wspec((H, H)), wspec((H, 1)), wspec((H, 1)), wspec((H, 1)),
            wspec((H, CI)), wspec((H, 1)), wspec((H, 1)), wspec((H, 1)),
            wspec((8, H)), wspec((8, 1)),
        ],
        out_specs=[
            pl.BlockSpec((1, H, TN), lambda b, n: (b, 0, n)),
            pl.BlockSpec((1, TN, H), lambda b, n: (b, n, 0)),
            pl.BlockSpec((1, 8, TN), lambda b, n: (b, 0, n)),
            pl.BlockSpec((1, TN, 8), lambda b, n: (b, n, 0)),
        ],
        out_shape=[
            jax.ShapeDtypeStruct((B, H, N), jnp.float32),
            jax.ShapeDtypeStruct((B, N, H), jnp.float32),
            jax.ShapeDtypeStruct((B, 8, N), jnp.float32),
            jax.ShapeDtypeStruct((B, N, 8), jnp.float32),
        ],
    )(f_sem, f_ins,
      W_sem, col(b_sem), col(g_sem), col(be_sem),
      W_ad, col(b_ad), col(g_ad), col(be_ad),
      W_ins, col(b_ins), col(g_ins), col(be_ins),
      W_ie8, col(b_ie8))
    return fs, fst, e8, e8t


def kernel(f_sem, f_ins, W_sem, b_sem, g_sem, be_sem, W_ad, b_ad, g_ad, be_ad,
           W_ins, b_ins, g_ins, be_ins, W_sp, b_sp, W_ie, b_ie, k):
    C_ie = W_ie.shape[0]
    W_ie8 = jnp.zeros((8, W_ie.shape[1]), jnp.float32).at[:C_ie].set(W_ie)
    b_ie8 = jnp.zeros((8,), jnp.float32).at[:C_ie].set(b_ie)

    fs, fst, e8, e8t = _mlps(f_sem, f_ins, W_sem, b_sem, g_sem, be_sem,
                             W_ad, b_ad, g_ad, be_ad,
                             W_ins, b_ins, g_ins, be_ins, W_ie8, b_ie8)
    B, H, N = fs.shape
    e_ins = e8[:, :C_ie, :]

    idx32 = _knn(e8, e8t)

    fst_flat = fst.reshape(B * N, H)
    idx_flat = idx32.reshape(B * N * KPAD)
    f_isemT = _sc_gather_max(B * N, H)(fst_flat, idx_flat)

    C_sp = W_sp.shape[0]
    W_sp16 = jnp.zeros((16, H), jnp.float32).at[:C_sp].set(W_sp)
    b_sp16 = jnp.zeros((16,), jnp.float32).at[:C_sp].set(b_sp).reshape(-1, 1)
    out = _psem(f_isemT, W_sp16, b_sp16)
    p_sem = out.reshape(B, -1, 16, out.shape[-1]).transpose(0, 2, 1, 3)
    p_sem = p_sem.reshape(B, 16, N)[:, :C_sp, :]
    return (p_sem, e_ins)


# speedup vs baseline: 23.3379x; 1.3197x over previous
import functools
import jax
import jax.numpy as jnp
from jax import lax
from jax.experimental import pallas as pl
from jax.experimental.pallas import tpu as pltpu
from jax.experimental.pallas import tpu_sc as plsc

EPS = 1e-5
K_STATIC = 30
KPAD = 32
LEVELS = 6
TR = 256
GR = 8
CN = 1024
INF = float('inf')
IMAX = 2**31 - 1
JBIG = 1.0e9


def _bf(x):
    return x.astype(jnp.bfloat16)


def _knn_kernel(e8_ref, e8t_ref, idx_out):
    b = pl.program_id(0)
    e8 = e8_ref[0]
    N = e8.shape[1]
    eb = e8.astype(jnp.bfloat16)
    sq = e8[0:1] * e8[0:1]
    for c in range(1, 5):
        sq = sq + e8[c:c+1] * e8[c:c+1]
    et = e8t_ref[0]
    ebt = et.astype(jnp.bfloat16)
    sqr = et[:, 0:1] * et[:, 0:1]
    for c in range(1, 5):
        sqr = sqr + et[:, c:c+1] * et[:, c:c+1]

    mv = [jnp.full((TR, 128), INF, jnp.float32) for _ in range(LEVELS)]
    mi = [jnp.full((TR, 128), JBIG, jnp.float32) for _ in range(LEVELS)]
    lane = jax.lax.broadcasted_iota(jnp.int32, (TR, 128), 1).astype(jnp.float32)

    for c in range(N // CN):
        ec = eb[:, c * CN:(c + 1) * CN]
        inner = jax.lax.dot_general(ebt, ec, (((1,), (0,)), ((), ())),
                                    preferred_element_type=jnp.float32)
        sqc = sq[:, c * CN:(c + 1) * CN]
        dist = (sqr + sqc) - 2.0 * inner
        for dd in range(CN // 128):
            v = dist[:, dd * 128:(dd + 1) * 128]
            j = lane + (c * CN + dd * 128)
            for l in range(LEVELS):
                lt = v < mv[l]
                nv = jnp.where(lt, v, mv[l])
                ni = jnp.where(lt, j, mi[l])
                v = jnp.where(lt, mv[l], v)
                j = jnp.where(lt, mi[l], j)
                mv[l], mi[l] = nv, ni

    NG = TR // GR
    gv = [[mv[l][g * GR:(g + 1) * GR] for l in range(LEVELS)]
          for g in range(NG)]
    gi = [[mi[l][g * GR:(g + 1) * GR] for l in range(LEVELS)]
          for g in range(NG)]
    outs = [[] for _ in range(NG)]
    for t in range(K_STATIC):
        for g in range(NG):
            win_v = jnp.min(gv[g][0], axis=1, keepdims=True)
            elig = gv[g][0] == win_v
            win_j = jnp.min(jnp.where(elig, gi[g][0], JBIG), axis=1,
                            keepdims=True)
            outs[g].append(win_j)
            is_win = gi[g][0] == win_j
            for l in range(LEVELS - 1):
                gv[g][l] = jnp.where(is_win, gv[g][l + 1], gv[g][l])
                gi[g][l] = jnp.where(is_win, gi[g][l + 1], gi[g][l])
            gv[g][LEVELS - 1] = jnp.where(is_win, INF, gv[g][LEVELS - 1])
            gi[g][LEVELS - 1] = jnp.where(is_win, JBIG, gi[g][LEVELS - 1])
    group_res = [jnp.concatenate(outs[g] + [outs[g][0], outs[g][0]], axis=1)
                 for g in range(NG)]
    idxf = jnp.concatenate(group_res, axis=0)
    idx_out[0] = idxf.astype(jnp.int32) + b * N


def _knn(e8, e8t):
    B, _, N = e8.shape
    return pl.pallas_call(
        _knn_kernel,
        grid=(B, N // TR),
        in_specs=[
            pl.BlockSpec((1, 8, N), lambda b, r: (b, 0, 0)),
            pl.BlockSpec((1, TR, 8), lambda b, r: (b, r, 0)),
        ],
        out_specs=pl.BlockSpec((1, TR, KPAD), lambda b, r: (b, r, 0)),
        out_shape=jax.ShapeDtypeStruct((B, N, KPAD), jnp.int32),
    )(e8, e8t)


def _sc_gather_max(NPTS, H):
    info = plsc.get_sparse_core_info()
    NC, NS, L = info.num_cores, info.num_subcores, info.num_lanes
    NW = NC * NS
    pts_per_w = NPTS // NW
    P = 8
    nchunk = pts_per_w // P
    NCH = H // L
    mesh = plsc.VectorSubcoreMesh(core_axis_name="c", subcore_axis_name="s")

    @functools.partial(
        pl.kernel, mesh=mesh,
        out_type=jax.ShapeDtypeStruct((NPTS, H), jnp.float32),
        scratch_types=[
            pltpu.VMEM((P * KPAD,), jnp.int32),
            pltpu.VMEM((P * KPAD,), jnp.int32),
            pltpu.VMEM((2, P * KPAD, H), jnp.float32),
            pltpu.VMEM((2, P, H), jnp.float32),
            pltpu.SemaphoreType.DMA,
            pltpu.SemaphoreType.DMA,
            pltpu.SemaphoreType.DMA,
            pltpu.SemaphoreType.DMA,
        ],
    )
    def sc_kernel(tab_hbm, idx_hbm, out_hbm, idx_c0, idx_c1, rows, acc,
                  gsem0, gsem1, osem0, osem1):
        wid = lax.axis_index("s") * NC + lax.axis_index("c")
        base = wid * pts_per_w
        gsems = (gsem0, gsem1)
        osems = (osem0, osem1)
        idx_cs = (idx_c0, idx_c1)

        def start_gather(ci, par):
            @pl.when(ci < nchunk)
            def _():
                p0 = base + ci * P
                pltpu.sync_copy(idx_hbm.at[pl.ds(p0 * KPAD, P * KPAD)],
                                idx_cs[par])
                pltpu.async_copy(tab_hbm.at[idx_cs[par]],
                                 rows.at[par], gsems[par])

        start_gather(0, 0)

        def compute(cj, ci, par):
            rv = rows.at[par]
            av = acc.at[par]
            p0 = base + ci * P
            pltpu.make_async_copy(tab_hbm.at[idx_cs[par]], rv,
                                  gsems[par]).wait()

            @pl.when(cj > 0)
            def _():
                pltpu.make_async_copy(av, out_hbm.at[pl.ds(p0, P)],
                                      osems[par]).wait()

            for p in range(P):
                accs = [rv[p * KPAD, pl.ds(ch * L, L)] for ch in range(NCH)]

                def k_body(kk, accs):
                    return tuple(
                        jnp.maximum(a, rv[p * KPAD + kk, pl.ds(ch * L, L)])
                        for ch, a in enumerate(accs))

                accs = lax.fori_loop(1, KPAD, k_body, tuple(accs))
                for ch in range(NCH):
                    av[p, pl.ds(ch * L, L)] = accs[ch]
            pltpu.async_copy(av, out_hbm.at[pl.ds(p0, P)], osems[par])

        def body2(cj, carry):
            ci0 = cj * 2
            start_gather(ci0 + 1, 1)
            compute(cj, ci0, 0)
            start_gather(ci0 + 2, 0)
            compute(cj, ci0 + 1, 1)
            return carry

        lax.fori_loop(0, nchunk // 2, body2, 0)
        pltpu.make_async_copy(acc.at[0], out_hbm.at[pl.ds(base, P)],
                              osem0).wait()
        pltpu.make_async_copy(acc.at[1], out_hbm.at[pl.ds(base, P)],
                              osem1).wait()

    return sc_kernel


def _psem_kernel(x_ref, w_ref, b_ref, out_ref):
    xt = x_ref[0].T
    out_ref[0] = (jnp.dot(_bf(w_ref[...]), _bf(xt),
                          preferred_element_type=jnp.float32) + b_ref[...])


def _psem(f_isemT, W_sp16, b_sp16):
    B_N, H = f_isemT.shape
    TN4 = 2048
    x = f_isemT.reshape(-1, TN4, H)
    G = x.shape[0]
    out = pl.pallas_call(
        _psem_kernel,
        grid=(G,),
        in_specs=[
            pl.BlockSpec((1, TN4, H), lambda g: (g, 0, 0)),
            pl.BlockSpec((16, H), lambda g: (0, 0)),
            pl.BlockSpec((16, 1), lambda g: (0, 0)),
        ],
        out_specs=pl.BlockSpec((1, 16, TN4), lambda g: (g, 0, 0)),
        out_shape=jax.ShapeDtypeStruct((G, 16, TN4), jnp.float32),
    )(x, W_sp16, b_sp16)
    return out


def _mlp_kernel(fsem_ref, fins_ref, wsem_ref, bsem_ref, gsem_ref, besem_ref,
                wad_ref, bad_ref, gad_ref, bead_ref,
                wins_ref, bins_ref, gins_ref, beins_ref,
                wie_ref, bie_ref,
                fs_out, fst_out, e8_out, e8t_out):
    s = jnp.sqrt(1.0 + EPS)

    def mlp(w_ref, x, b_ref, g_ref, be_ref):
        y = jnp.dot(_bf(w_ref[...]), _bf(x),
                    preferred_element_type=jnp.float32) + b_ref[...]
        y = y / s * g_ref[...] + be_ref[...]
        return jnp.maximum(y, 0.0)

    fs = mlp(wsem_ref, fsem_ref[0], bsem_ref, gsem_ref, besem_ref)
    adapted = mlp(wad_ref, fs, bad_ref, gad_ref, bead_ref)
    fi = mlp(wins_ref, fins_ref[0], bins_ref, gins_ref, beins_ref)
    f_sins = fi + adapted
    e8 = jnp.dot(_bf(wie_ref[...]), _bf(f_sins),
                 preferred_element_type=jnp.float32) + bie_ref[...]
    fs_out[0] = fs
    fst_out[0] = fs.T
    e8_out[0] = e8
    e8t_out[0] = e8.T


def _mlps(f_sem, f_ins, W_sem, b_sem, g_sem, be_sem, W_ad, b_ad, g_ad, be_ad,
          W_ins, b_ins, g_ins, be_ins, W_ie8, b_ie8):
    B, CI, N = f_sem.shape
    H = W_sem.shape[0]
    TN = 1024
    col = lambda v: v.reshape(-1, 1)
    grid = (B, N // TN)
    wspec = lambda shp: pl.BlockSpec(shp, lambda b, n: (0, 0))
    fs, fst, e8, e8t = pl.pallas_call(
        _mlp_kernel,
        grid=grid,
        in_specs=[
            pl.BlockSpec((1, CI, TN), lambda b, n: (b, 0, n)),
            pl.BlockSpec((1, CI, TN), lambda b, n: (b, 0, n)),
            wspec((H, CI)), wspec((H, 1)), wspec((H, 1)), wspec((H, 1)),
            wspec((H, H)), wspec((H, 1)), wspec((H, 1)), wspec((H, 1)),
            wspec((H, CI)), wspec((H, 1)), wspec((H, 1)), wspec((H, 1)),
            wspec((8, H)), wspec((8, 1)),
        ],
        out_specs=[
            pl.BlockSpec((1, H, TN), lambda b, n: (b, 0, n)),
            pl.BlockSpec((1, TN, H), lambda b, n: (b, n, 0)),
            pl.BlockSpec((1, 8, TN), lambda b, n: (b, 0, n)),
            pl.BlockSpec((1, TN, 8), lambda b, n: (b, n, 0)),
        ],
        out_shape=[
            jax.ShapeDtypeStruct((B, H, N), jnp.float32),
            jax.ShapeDtypeStruct((B, N, H), jnp.float32),
            jax.ShapeDtypeStruct((B, 8, N), jnp.float32),
            jax.ShapeDtypeStruct((B, N, 8), jnp.float32),
        ],
    )(f_sem, f_ins,
      W_sem, col(b_sem), col(g_sem), col(be_sem),
      W_ad, col(b_ad), col(g_ad), col(be_ad),
      W_ins, col(b_ins), col(g_ins), col(be_ins),
      W_ie8, col(b_ie8))
    return fs, fst, e8, e8t


def kernel(f_sem, f_ins, W_sem, b_sem, g_sem, be_sem, W_ad, b_ad, g_ad, be_ad,
           W_ins, b_ins, g_ins, be_ins, W_sp, b_sp, W_ie, b_ie, k):
    C_ie = W_ie.shape[0]
    W_ie8 = jnp.zeros((8, W_ie.shape[1]), jnp.float32).at[:C_ie].set(W_ie)
    b_ie8 = jnp.zeros((8,), jnp.float32).at[:C_ie].set(b_ie)

    fs, fst, e8, e8t = _mlps(f_sem, f_ins, W_sem, b_sem, g_sem, be_sem,
                             W_ad, b_ad, g_ad, be_ad,
                             W_ins, b_ins, g_ins, be_ins, W_ie8, b_ie8)
    B, H, N = fs.shape
    e_ins = e8[:, :C_ie, :]

    idx32 = _knn(e8, e8t)

    fst_flat = fst.reshape(B * N, H)
    idx_flat = idx32.reshape(B * N * KPAD)
    f_isemT = _sc_gather_max(B * N, H)(fst_flat, idx_flat)

    C_sp = W_sp.shape[0]
    W_sp16 = jnp.zeros((16, H), jnp.float32).at[:C_sp].set(W_sp)
    b_sp16 = jnp.zeros((16,), jnp.float32).at[:C_sp].set(b_sp).reshape(-1, 1)
    out = _psem(f_isemT, W_sp16, b_sp16)
    p_sem = out.reshape(B, -1, 16, out.shape[-1]).transpose(0, 2, 1, 3)
    p_sem = p_sem.reshape(B, 16, N)[:, :C_sp, :]
    return (p_sem, e_ins)


# speedup vs baseline: 24.5892x; 1.0536x over previous
import functools
import jax
import jax.numpy as jnp
from jax import lax
from jax.experimental import pallas as pl
from jax.experimental.pallas import tpu as pltpu
from jax.experimental.pallas import tpu_sc as plsc

EPS = 1e-5
K_STATIC = 30
KPAD = 32
LEVELS = 6
TR = 256
GR = 8
CN = 1024
INF = float('inf')
IMAX = 2**31 - 1
JBIG = 1.0e9


def _bf(x):
    return x.astype(jnp.bfloat16)


def _knn_kernel(e8_ref, e8t_ref, idx_out, *, b_off):
    b = pl.program_id(0) + b_off
    e8 = e8_ref[0]
    N = e8.shape[1]
    eb = e8.astype(jnp.bfloat16)
    sq = e8[0:1] * e8[0:1]
    for c in range(1, 5):
        sq = sq + e8[c:c+1] * e8[c:c+1]
    et = e8t_ref[0]
    ebt = et.astype(jnp.bfloat16)
    sqr = et[:, 0:1] * et[:, 0:1]
    for c in range(1, 5):
        sqr = sqr + et[:, c:c+1] * et[:, c:c+1]

    mv = [jnp.full((TR, 128), INF, jnp.float32) for _ in range(LEVELS)]
    mi = [jnp.full((TR, 128), JBIG, jnp.float32) for _ in range(LEVELS)]
    lane = jax.lax.broadcasted_iota(jnp.int32, (TR, 128), 1).astype(jnp.float32)

    for c in range(N // CN):
        ec = eb[:, c * CN:(c + 1) * CN]
        inner = jax.lax.dot_general(ebt, ec, (((1,), (0,)), ((), ())),
                                    preferred_element_type=jnp.float32)
        sqc = sq[:, c * CN:(c + 1) * CN]
        dist = (sqr + sqc) - 2.0 * inner
        for dd in range(CN // 128):
            v = dist[:, dd * 128:(dd + 1) * 128]
            j = lane + (c * CN + dd * 128)
            for l in range(LEVELS):
                lt = v < mv[l]
                nv = jnp.where(lt, v, mv[l])
                ni = jnp.where(lt, j, mi[l])
                v = jnp.where(lt, mv[l], v)
                j = jnp.where(lt, mi[l], j)
                mv[l], mi[l] = nv, ni

    NG = TR // GR
    gv = [[mv[l][g * GR:(g + 1) * GR] for l in range(LEVELS)]
          for g in range(NG)]
    gi = [[mi[l][g * GR:(g + 1) * GR] for l in range(LEVELS)]
          for g in range(NG)]
    outs = [[] for _ in range(NG)]
    for t in range(K_STATIC):
        for g in range(NG):
            win_v = jnp.min(gv[g][0], axis=1, keepdims=True)
            elig = gv[g][0] == win_v
            win_j = jnp.min(jnp.where(elig, gi[g][0], JBIG), axis=1,
                            keepdims=True)
            outs[g].append(win_j)
            is_win = gi[g][0] == win_j
            for l in range(LEVELS - 1):
                gv[g][l] = jnp.where(is_win, gv[g][l + 1], gv[g][l])
                gi[g][l] = jnp.where(is_win, gi[g][l + 1], gi[g][l])
            gv[g][LEVELS - 1] = jnp.where(is_win, INF, gv[g][LEVELS - 1])
            gi[g][LEVELS - 1] = jnp.where(is_win, JBIG, gi[g][LEVELS - 1])
    group_res = [jnp.concatenate(outs[g] + [outs[g][0], outs[g][0]], axis=1)
                 for g in range(NG)]
    idxf = jnp.concatenate(group_res, axis=0)
    idx_out[0] = idxf.astype(jnp.int32) + b * N


def _knn(e8, e8t, b_off=0):
    B, _, N = e8.shape
    return pl.pallas_call(
        functools.partial(_knn_kernel, b_off=b_off),
        grid=(B, N // TR),
        in_specs=[
            pl.BlockSpec((1, 8, N), lambda b, r: (b, 0, 0)),
            pl.BlockSpec((1, TR, 8), lambda b, r: (b, r, 0)),
        ],
        out_specs=pl.BlockSpec((1, TR, KPAD), lambda b, r: (b, r, 0)),
        out_shape=jax.ShapeDtypeStruct((B, N, KPAD), jnp.int32),
    )(e8, e8t)


def _sc_gather_max(NPTS, H):
    info = plsc.get_sparse_core_info()
    NC, NS, L = info.num_cores, info.num_subcores, info.num_lanes
    NW = NC * NS
    pts_per_w = NPTS // NW
    P = 8
    nchunk = pts_per_w // P
    NCH = H // L
    mesh = plsc.VectorSubcoreMesh(core_axis_name="c", subcore_axis_name="s")

    @functools.partial(
        pl.kernel, mesh=mesh,
        out_type=jax.ShapeDtypeStruct((NPTS, H), jnp.float32),
        scratch_types=[
            pltpu.VMEM((P * KPAD,), jnp.int32),
            pltpu.VMEM((P * KPAD,), jnp.int32),
            pltpu.VMEM((2, P * KPAD, H), jnp.float32),
            pltpu.VMEM((2, P, H), jnp.float32),
            pltpu.SemaphoreType.DMA,
            pltpu.SemaphoreType.DMA,
            pltpu.SemaphoreType.DMA,
            pltpu.SemaphoreType.DMA,
        ],
    )
    def sc_kernel(tab_hbm, idx_hbm, out_hbm, idx_c0, idx_c1, rows, acc,
                  gsem0, gsem1, osem0, osem1):
        wid = lax.axis_index("s") * NC + lax.axis_index("c")
        base = wid * pts_per_w
        gsems = (gsem0, gsem1)
        osems = (osem0, osem1)
        idx_cs = (idx_c0, idx_c1)

        def start_gather(ci, par):
            @pl.when(ci < nchunk)
            def _():
                p0 = base + ci * P
                pltpu.sync_copy(idx_hbm.at[pl.ds(p0 * KPAD, P * KPAD)],
                                idx_cs[par])
                pltpu.async_copy(tab_hbm.at[idx_cs[par]],
                                 rows.at[par], gsems[par])

        start_gather(0, 0)

        def compute(cj, ci, par):
            rv = rows.at[par]
            av = acc.at[par]
            p0 = base + ci * P
            pltpu.make_async_copy(tab_hbm.at[idx_cs[par]], rv,
                                  gsems[par]).wait()

            @pl.when(cj > 0)
            def _():
                pltpu.make_async_copy(av, out_hbm.at[pl.ds(p0, P)],
                                      osems[par]).wait()

            for p in range(P):
                accs = [rv[p * KPAD, pl.ds(ch * L, L)] for ch in range(NCH)]

                def k_body(kk, accs):
                    r0 = 1 + kk * 3
                    for o in range(3):
                        accs = tuple(
                            jnp.maximum(a, rv[p * KPAD + r0 + o,
                                              pl.ds(ch * L, L)])
                            for ch, a in enumerate(accs))
                    return accs

                accs = lax.fori_loop(0, 10, k_body, tuple(accs))
                accs = tuple(
                    jnp.maximum(a, rv[p * KPAD + 31, pl.ds(ch * L, L)])
                    for ch, a in enumerate(accs))
                for ch in range(NCH):
                    av[p, pl.ds(ch * L, L)] = accs[ch]
            pltpu.async_copy(av, out_hbm.at[pl.ds(p0, P)], osems[par])

        def body2(cj, carry):
            ci0 = cj * 2
            start_gather(ci0 + 1, 1)
            compute(cj, ci0, 0)
            start_gather(ci0 + 2, 0)
            compute(cj, ci0 + 1, 1)
            return carry

        lax.fori_loop(0, nchunk // 2, body2, 0)
        pltpu.make_async_copy(acc.at[0], out_hbm.at[pl.ds(base, P)],
                              osem0).wait()
        pltpu.make_async_copy(acc.at[1], out_hbm.at[pl.ds(base, P)],
                              osem1).wait()

    return sc_kernel


def _psem_kernel(x_ref, w_ref, b_ref, out_ref):
    xt = x_ref[0].T
    out_ref[0] = (jnp.dot(_bf(w_ref[...]), _bf(xt),
                          preferred_element_type=jnp.float32) + b_ref[...])


def _psem(f_isemT, W_sp16, b_sp16):
    B_N, H = f_isemT.shape
    TN4 = 2048
    x = f_isemT.reshape(-1, TN4, H)
    G = x.shape[0]
    out = pl.pallas_call(
        _psem_kernel,
        grid=(G,),
        in_specs=[
            pl.BlockSpec((1, TN4, H), lambda g: (g, 0, 0)),
            pl.BlockSpec((16, H), lambda g: (0, 0)),
            pl.BlockSpec((16, 1), lambda g: (0, 0)),
        ],
        out_specs=pl.BlockSpec((1, 16, TN4), lambda g: (g, 0, 0)),
        out_shape=jax.ShapeDtypeStruct((G, 16, TN4), jnp.float32),
    )(x, W_sp16, b_sp16)
    return out


def _mlp_kernel(fsem_ref, fins_ref, wsem_ref, bsem_ref, gsem_ref, besem_ref,
                wad_ref, bad_ref, gad_ref, bead_ref,
                wins_ref, bins_ref, gins_ref, beins_ref,
                wie_ref, bie_ref,
                fs_out, fst_out, e8_out, e8t_out):
    s = jnp.sqrt(1.0 + EPS)

    def mlp(w_ref, x, b_ref, g_ref, be_ref):
        y = jnp.dot(_bf(w_ref[...]), _bf(x),
                    preferred_element_type=jnp.float32) + b_ref[...]
        y = y / s * g_ref[...] + be_ref[...]
        return jnp.maximum(y, 0.0)

    fs = mlp(wsem_ref, fsem_ref[0], bsem_ref, gsem_ref, besem_ref)
    adapted = mlp(wad_ref, fs, bad_ref, gad_ref, bead_ref)
    fi = mlp(wins_ref, fins_ref[0], bins_ref, gins_ref, beins_ref)
    f_sins = fi + adapted
    e8 = jnp.dot(_bf(wie_ref[...]), _bf(f_sins),
                 preferred_element_type=jnp.float32) + bie_ref[...]
    fs_out[0] = fs
    fst_out[0] = fs.T
    e8_out[0] = e8
    e8t_out[0] = e8.T


def _mlps(f_sem, f_ins, W_sem, b_sem, g_sem, be_sem, W_ad, b_ad, g_ad, be_ad,
          W_ins, b_ins, g_ins, be_ins, W_ie8, b_ie8):
    B, CI, N = f_sem.shape
    H = W_sem.shape[0]
    TN = 1024
    col = lambda v: v.reshape(-1, 1)
    grid = (B, N // TN)
    wspec = lambda shp: pl.BlockSpec(shp, lambda b, n: (0, 0))
    fs, fst, e8, e8t = pl.pallas_call(
        _mlp_kernel,
        grid=grid,
        in_specs=[
            pl.BlockSpec((1, CI, TN), lambda b, n: (b, 0, n)),
            pl.BlockSpec((1, CI, TN), lambda b, n: (b, 0, n)),
            wspec((H, CI)), wspec((H, 1)), wspec((H, 1)), wspec((H, 1)),
            wspec((H, H)), wspec((H, 1)), wspec((H, 1)), wspec((H, 1)),
            wspec((H, CI)), wspec((H, 1)), wspec((H, 1)), wspec((H, 1)),
            wspec((8, H)), wspec((8, 1)),
        ],
        out_specs=[
            pl.BlockSpec((1, H, TN), lambda b, n: (b, 0, n)),
            pl.BlockSpec((1, TN, H), lambda b, n: (b, n, 0)),
            pl.BlockSpec((1, 8, TN), lambda b, n: (b, 0, n)),
            pl.BlockSpec((1, TN, 8), lambda b, n: (b, n, 0)),
        ],
        out_shape=[
            jax.ShapeDtypeStruct((B, H, N), jnp.float32),
            jax.ShapeDtypeStruct((B, N, H), jnp.float32),
            jax.ShapeDtypeStruct((B, 8, N), jnp.float32),
            jax.ShapeDtypeStruct((B, N, 8), jnp.float32),
        ],
    )(f_sem, f_ins,
      W_sem, col(b_sem), col(g_sem), col(be_sem),
      W_ad, col(b_ad), col(g_ad), col(be_ad),
      W_ins, col(b_ins), col(g_ins), col(be_ins),
      W_ie8, col(b_ie8))
    return fs, fst, e8, e8t


def kernel(f_sem, f_ins, W_sem, b_sem, g_sem, be_sem, W_ad, b_ad, g_ad, be_ad,
           W_ins, b_ins, g_ins, be_ins, W_sp, b_sp, W_ie, b_ie, k):
    C_ie = W_ie.shape[0]
    W_ie8 = jnp.zeros((8, W_ie.shape[1]), jnp.float32).at[:C_ie].set(W_ie)
    b_ie8 = jnp.zeros((8,), jnp.float32).at[:C_ie].set(b_ie)

    fs, fst, e8, e8t = _mlps(f_sem, f_ins, W_sem, b_sem, g_sem, be_sem,
                             W_ad, b_ad, g_ad, be_ad,
                             W_ins, b_ins, g_ins, be_ins, W_ie8, b_ie8)
    B, H, N = fs.shape
    e_ins = e8[:, :C_ie, :]

    fst_flat = fst.reshape(B * N, H)
    BH = B // 2
    sc_call = _sc_gather_max(BH * N, H)
    halves = []
    for h in range(2):
        sl = slice(h * BH, (h + 1) * BH)
        idx32 = _knn(e8[sl], e8t[sl], b_off=h * BH)
        halves.append(sc_call(fst_flat, idx32.reshape(BH * N * KPAD)))
    f_isemT = jnp.concatenate(halves, axis=0)

    C_sp = W_sp.shape[0]
    W_sp16 = jnp.zeros((16, H), jnp.float32).at[:C_sp].set(W_sp)
    b_sp16 = jnp.zeros((16,), jnp.float32).at[:C_sp].set(b_sp).reshape(-1, 1)
    out = _psem(f_isemT, W_sp16, b_sp16)
    p_sem = out.reshape(B, -1, 16, out.shape[-1]).transpose(0, 2, 1, 3)
    p_sem = p_sem.reshape(B, 16, N)[:, :C_sp, :]
    return (p_sem, e_ins)
